# baseline (speedup 1.0000x reference)
import functools

import jax
import jax.numpy as jnp
from jax import lax
from jax.experimental import pallas as pl
from jax.experimental.pallas import tpu as pltpu
from jax.experimental.pallas import tpu_sc as plsc

_NC = 2
_NS = 16
_NW = _NC * _NS
_BB = 128


@functools.partial(jax.jit, static_argnums=(2,))
def _emb_lookup(x, weight, L):
    B = x.shape[0]
    D = weight.shape[1]
    TR = D // 8
    mesh = plsc.VectorSubcoreMesh(core_axis_name="c", subcore_axis_name="s")

    @functools.partial(
        pl.kernel,
        out_type=jax.ShapeDtypeStruct((L, TR, B // _BB, 8, 128), jnp.float32),
        mesh=mesh,
        compiler_params=pltpu.CompilerParams(
            use_tc_tiling_on_sc=False, needs_layout_passes=False),
        scratch_types=[
            pltpu.VMEM((_BB, L), jnp.int32),
            pltpu.VMEM((L, _BB), jnp.int32),
            pltpu.VMEM((2, _BB, D), jnp.float32),
            pltpu.VMEM((2, TR, 8, 128), jnp.float32),
            pltpu.SemaphoreType.DMA,
            pltpu.SemaphoreType.DMA,
            pltpu.SemaphoreType.DMA,
            pltpu.SemaphoreType.DMA,
        ],
    )
    def body(x_hbm, w_hbm, out_hbm, idx_bl, idx_lb, rows_v, tiles_v,
             g0, g1, w0, w1):
        gsem = [g0, g1]
        wsem = [w0, w1]
        wid = lax.axis_index("s") * _NC + lax.axis_index("c")
        pltpu.sync_copy(x_hbm.at[pl.ds(wid * _BB, _BB)], idx_bl)

        @pl.loop(0, L)
        def tpose_idx(l):
            col = jnp.full((16,), l, jnp.int32)
            for g in range(_BB // 16):
                ridx = lax.iota(jnp.int32, 16) + 16 * g
                idx_lb[l, pl.ds(16 * g, 16)] = plsc.load_gather(
                    idx_bl, [ridx, col])

        def fire_gather(l, sub):
            pltpu.async_copy(
                w_hbm.at[idx_lb.at[l]], rows_v.at[sub], gsem[sub])

        def wait_gather(sub):
            pltpu.make_async_copy(
                w_hbm.at[idx_lb.at[0]], rows_v.at[sub], gsem[sub]).wait()

        def fire_write(l, sub):
            pltpu.async_copy(
                tiles_v.at[sub], out_hbm.at[l, :, wid], wsem[sub])

        def wait_write(sub):
            pltpu.make_async_copy(
                tiles_v.at[sub], out_hbm.at[0, :, wid], wsem[sub]).wait()

        def transpose(sub):
            @pl.loop(0, TR)
            def ttr(tr):
                for r in range(8):
                    col = jnp.full((16,), tr * 8 + r, jnp.int32)
                    for g in range(_BB // 16):
                        ridx = lax.iota(jnp.int32, 16) + 16 * g
                        tiles_v[sub, tr, r, pl.ds(16 * g, 16)] = (
                            plsc.load_gather(rows_v.at[sub], [ridx, col]))

        fire_gather(0, 0)
        fire_gather(1, 1)

        @pl.loop(0, L // 2)
        def lp(i):
            for sub in range(2):
                l = 2 * i + sub
                wait_gather(sub)

                @pl.when(i > 0)
                def _():
                    wait_write(sub)

                transpose(sub)
                fire_write(l, sub)

                @pl.when(i < L // 2 - 1)
                def _():
                    fire_gather(l + 2, sub)

        wait_write(0)
        wait_write(1)

    return body(x, weight)


def kernel(x, weight):
    B, L = x.shape
    V, D = weight.shape
    assert B % (_NW * _BB) == 0 or B == _NW * _BB
    assert D % 8 == 0 and L % 2 == 0
    out5 = _emb_lookup(x.astype(jnp.int32), weight, L)
    return out5.transpose((2, 4, 0, 1, 3)).reshape(B, L, D)


# speedup vs baseline: 1.5812x; 1.5812x over previous
import functools

import jax
import jax.numpy as jnp
from jax import lax
from jax.experimental import pallas as pl
from jax.experimental.pallas import tpu as pltpu
from jax.experimental.pallas import tpu_sc as plsc

_NC = 2
_NS = 16
_NW = _NC * _NS
_BB = 128


@functools.partial(jax.jit, static_argnums=(2,))
def _emb_lookup(x, weight, L):
    B = x.shape[0]
    D = weight.shape[1]
    TR = D // 8
    mesh = plsc.VectorSubcoreMesh(core_axis_name="c", subcore_axis_name="s")

    @functools.partial(
        pl.kernel,
        out_type=jax.ShapeDtypeStruct((L, TR, B // _BB, 8, 128), jnp.float32),
        mesh=mesh,
        compiler_params=pltpu.CompilerParams(
            use_tc_tiling_on_sc=False, needs_layout_passes=False),
        scratch_types=[
            pltpu.VMEM((_BB, L), jnp.int32),
            pltpu.VMEM((L, _BB), jnp.int32),
            pltpu.VMEM((2, _BB, D), jnp.float32),
            pltpu.VMEM((2, TR, 8, 128), jnp.float32),
            pltpu.SemaphoreType.DMA,
            pltpu.SemaphoreType.DMA,
            pltpu.SemaphoreType.DMA,
            pltpu.SemaphoreType.DMA,
        ],
    )
    def body(x_hbm, w_hbm, out_hbm, idx_bl, idx_lb, rows_v, tiles_v,
             g0, g1, w0, w1):
        gsem = [g0, g1]
        wsem = [w0, w1]
        wid = lax.axis_index("s") * _NC + lax.axis_index("c")
        pltpu.sync_copy(x_hbm.at[pl.ds(wid * _BB, _BB)], idx_bl)

        iota = lax.iota(jnp.int32, 16)
        rot = [(iota + k) & 15 for k in range(16)]

        @pl.loop(0, L // 16)
        def tpose_idx(gl):
            for gb in range(_BB // 16):
                rix = iota + 16 * gb
                for k in range(16):
                    cix = 16 * gl + rot[k]
                    v = plsc.load_gather(idx_bl, [rix, cix])
                    plsc.store_scatter(idx_lb, [cix, rix], v)

        if L % 16:
            for gb in range(_BB // 16):
                rix = iota + 16 * gb
                for k in range(16):
                    cix = (L // 16) * 16 + rot[k]
                    msk = cix < L
                    v = plsc.load_gather(idx_bl, [rix, cix], mask=msk)
                    plsc.store_scatter(idx_lb, [cix, rix], v, mask=msk)

        def fire_gather(l, sub):
            pltpu.async_copy(
                w_hbm.at[idx_lb.at[l]], rows_v.at[sub], gsem[sub])

        def wait_gather(sub):
            pltpu.make_async_copy(
                w_hbm.at[idx_lb.at[0]], rows_v.at[sub], gsem[sub]).wait()

        def fire_write(l, sub):
            pltpu.async_copy(
                tiles_v.at[sub], out_hbm.at[l, :, wid], wsem[sub])

        def wait_write(sub):
            pltpu.make_async_copy(
                tiles_v.at[sub], out_hbm.at[0, :, wid], wsem[sub]).wait()

        def transpose(sub):
            rv = rows_v.at[sub]
            tv = tiles_v.at[sub]

            @pl.loop(0, _BB // 16)
            def tb(gb):
                rix = iota + 16 * gb
                for gd in range(D // 16):
                    for k in range(16):
                        cix = 16 * gd + rot[k]
                        v = plsc.load_gather(rv, [rix, cix])
                        plsc.store_scatter(
                            tv, [cix >> 3, cix & 7, rix], v)

        fire_gather(0, 0)
        fire_gather(1, 1)

        @pl.loop(0, L // 2)
        def lp(i):
            for sub in range(2):
                l = 2 * i + sub
                wait_gather(sub)

                @pl.when(i > 0)
                def _():
                    wait_write(sub)

                transpose(sub)
                fire_write(l, sub)

                @pl.when(i < L // 2 - 1)
                def _():
                    fire_gather(l + 2, sub)

        wait_write(0)
        wait_write(1)

    return body(x, weight)


def kernel(x, weight):
    B, L = x.shape
    V, D = weight.shape
    assert B % (_NW * _BB) == 0 or B == _NW * _BB
    assert D % 8 == 0 and L % 2 == 0
    out5 = _emb_lookup(x.astype(jnp.int32), weight, L)
    return out5.transpose((2, 4, 0, 1, 3)).reshape(B, L, D)
